# baseline (speedup 1.0000x reference)
import functools

import jax
import jax.numpy as jnp
from jax import lax
from jax.experimental import pallas as pl
from jax.experimental.pallas import tpu as pltpu
from jax.experimental.pallas import tpu_sc as plsc

BIGRAM_VOCAB = 1000000
BIGRAM_DIM = 32
MODEL_DIM = 1024
MOD = BIGRAM_VOCAB - 1
SEQ = 4096

_info = plsc.get_sparse_core_info()
NC, NS, L = _info.num_cores, _info.num_subcores, _info.num_lanes
NW = NC * NS

N_TOK = 16384
TOK_PER_W = N_TOK // NW
VECS_PER_W = TOK_PER_W // 16
GCHUNK = 128
NGC = TOK_PER_W // GCHUNK


def _sc_hash_gather(tok_hbm, table_hbm, emb_hbm, tok_v, h_v, rows_v, sem):
    wid = lax.axis_index("s") * NC + lax.axis_index("c")
    base = pl.multiple_of(wid * TOK_PER_W, TOK_PER_W)

    pltpu.sync_copy(tok_hbm.at[pl.ds(base, TOK_PER_W)],
                    tok_v.at[pl.ds(16, TOK_PER_W)])
    prev_off = pl.multiple_of(jnp.maximum(base - 8, 0), 8)
    pltpu.sync_copy(tok_hbm.at[pl.ds(prev_off, 8)], tok_v.at[pl.ds(8, 8)])

    seq_phase = base % SEQ
    lane = lax.iota(jnp.int32, 16)

    for j in range(VECS_PER_W):
        cur = tok_v[pl.ds(16 + 16 * j, 16)]
        prv = tok_v[pl.ds(15 + 16 * j, 16)]
        h = lax.rem(lax.bitwise_xor(cur * 36313, prv * 27191),
                    jnp.int32(MOD))
        if j == 0:
            h = jnp.where(lane + seq_phase == 0, jnp.int32(MOD), h)
        c, r = j // 8, (j % 8) * 16
        h_v[c, pl.ds(r, 16)] = h

    copies = [
        pltpu.async_copy(table_hbm.at[h_v.at[c]],
                         rows_v.at[pl.ds(c * GCHUNK, GCHUNK), :], sem)
        for c in range(NGC)
    ]
    for cp in copies:
        cp.wait()

    pltpu.sync_copy(rows_v, emb_hbm.at[pl.ds(base, TOK_PER_W), :])


def _sc_gather(tok_flat, embed_table):
    mesh = plsc.VectorSubcoreMesh(core_axis_name="c", subcore_axis_name="s")
    k = functools.partial(
        pl.kernel,
        mesh=mesh,
        out_type=jax.ShapeDtypeStruct((N_TOK, BIGRAM_DIM), jnp.float32),
        scratch_types=[
            pltpu.VMEM((TOK_PER_W + 16,), jnp.int32),
            pltpu.VMEM((NGC, GCHUNK), jnp.int32),
            pltpu.VMEM((TOK_PER_W, BIGRAM_DIM), jnp.float32),
            pltpu.SemaphoreType.DMA,
        ],
        compiler_params=pltpu.CompilerParams(use_tc_tiling_on_sc=False),
    )(_sc_hash_gather)
    return k(tok_flat, embed_table)


def _proj_body(scale_ref, emb_ref, w_ref, out_ref):
    acc = lax.dot_general(emb_ref[...], w_ref[...],
                          dimension_numbers=(((1,), (1,)), ((), ())),
                          preferred_element_type=jnp.float32)
    out_ref[...] = acc * scale_ref[0]


def _project(emb, proj_W, scale):
    rows_blk = 512
    grid = (N_TOK // rows_blk,)
    return pl.pallas_call(
        _proj_body,
        grid=grid,
        in_specs=[
            pl.BlockSpec(memory_space=pltpu.SMEM),
            pl.BlockSpec((rows_blk, BIGRAM_DIM), lambda i: (i, 0)),
            pl.BlockSpec((MODEL_DIM, BIGRAM_DIM), lambda i: (0, 0)),
        ],
        out_specs=pl.BlockSpec((rows_blk, MODEL_DIM), lambda i: (i, 0)),
        out_shape=jax.ShapeDtypeStruct((N_TOK, MODEL_DIM), jnp.float32),
    )(scale, emb, proj_W)


def kernel(token_ids, embed_table, proj_W, bigram_scale):
    b, s = token_ids.shape
    tok_flat = token_ids.astype(jnp.int32).reshape(-1)
    emb = _sc_gather(tok_flat, embed_table)
    scale = bigram_scale.astype(jnp.float32).reshape(1)
    out = _project(emb, proj_W, scale)
    return out.reshape(b, s, MODEL_DIM)


# speedup vs baseline: 1.0023x; 1.0023x over previous
import functools

import jax
import jax.numpy as jnp
from jax import lax
from jax.experimental import pallas as pl
from jax.experimental.pallas import tpu as pltpu
from jax.experimental.pallas import tpu_sc as plsc

BIGRAM_VOCAB = 1000000
BIGRAM_DIM = 32
MODEL_DIM = 1024
MOD = BIGRAM_VOCAB - 1
SEQ = 4096
PACK = 128 // BIGRAM_DIM

_info = plsc.get_sparse_core_info()
NC, NS, L = _info.num_cores, _info.num_subcores, _info.num_lanes
NW = NC * NS

N_TOK = 16384
TOK_PER_W = N_TOK // NW
VECS_PER_W = TOK_PER_W // 16
GCHUNK = 128
NGC = TOK_PER_W // GCHUNK


def _sc_hash_gather(tok_hbm, table_hbm, emb_hbm, q_hbm, tok_v, h_v, q_v,
                    rows_v, sem):
    wid = lax.axis_index("s") * NC + lax.axis_index("c")
    base = pl.multiple_of(wid * TOK_PER_W, TOK_PER_W)

    pltpu.sync_copy(tok_hbm.at[pl.ds(base, TOK_PER_W)],
                    tok_v.at[pl.ds(16, TOK_PER_W)])
    prev_off = pl.multiple_of(jnp.maximum(base - 8, 0), 8)
    pltpu.sync_copy(tok_hbm.at[pl.ds(prev_off, 8)], tok_v.at[pl.ds(8, 8)])

    seq_phase = base % SEQ
    lane = lax.iota(jnp.int32, 16)

    for j in range(VECS_PER_W):
        cur = tok_v[pl.ds(16 + 16 * j, 16)]
        prv = tok_v[pl.ds(15 + 16 * j, 16)]
        h = lax.rem(lax.bitwise_xor(cur * 36313, prv * 27191),
                    jnp.int32(MOD))
        if j == 0:
            h = jnp.where(lane + seq_phase == 0, jnp.int32(MOD), h)
        c, r = j // 8, (j % 8) * 16
        h_v[c, pl.ds(r, 16)] = lax.shift_right_logical(h, 2)
        q_v[pl.ds(16 * j, 16)] = lax.bitwise_and(h, 3)

    copies = [
        pltpu.async_copy(table_hbm.at[h_v.at[c]],
                         rows_v.at[pl.ds(c * GCHUNK, GCHUNK), :], sem)
        for c in range(NGC)
    ]
    for cp in copies:
        cp.wait()

    pltpu.sync_copy(rows_v, emb_hbm.at[pl.ds(base, TOK_PER_W), :])
    pltpu.sync_copy(q_v, q_hbm.at[pl.ds(base, TOK_PER_W)])


def _sc_gather(tok_flat, table4):
    mesh = plsc.VectorSubcoreMesh(core_axis_name="c", subcore_axis_name="s")
    k = functools.partial(
        pl.kernel,
        mesh=mesh,
        out_type=(
            jax.ShapeDtypeStruct((N_TOK, 128), jnp.float32),
            jax.ShapeDtypeStruct((N_TOK,), jnp.int32),
        ),
        scratch_types=[
            pltpu.VMEM((TOK_PER_W + 16,), jnp.int32),
            pltpu.VMEM((NGC, GCHUNK), jnp.int32),
            pltpu.VMEM((TOK_PER_W,), jnp.int32),
            pltpu.VMEM((TOK_PER_W, 128), jnp.float32),
            pltpu.SemaphoreType.DMA,
        ],
    )(_sc_hash_gather)
    return k(tok_flat, table4)


def _proj_body(scale_ref, emb_ref, q_ref, w_ref, out_ref):
    q = q_ref[0, 0, :]
    grp = lax.broadcasted_iota(jnp.int32, emb_ref.shape, 1) // BIGRAM_DIM
    sel = jnp.where(grp == q[:, None], emb_ref[...], 0.0)
    acc = lax.dot_general(sel, w_ref[...],
                          dimension_numbers=(((1,), (1,)), ((), ())),
                          preferred_element_type=jnp.float32)
    out_ref[...] = acc * scale_ref[0]


def _project(emb4, q3, proj_W4, scale):
    rows_blk = 512
    grid = (N_TOK // rows_blk,)
    return pl.pallas_call(
        _proj_body,
        grid=grid,
        in_specs=[
            pl.BlockSpec(memory_space=pltpu.SMEM),
            pl.BlockSpec((rows_blk, 128), lambda i: (i, 0)),
            pl.BlockSpec((1, 1, rows_blk), lambda i: (i, 0, 0)),
            pl.BlockSpec((MODEL_DIM, 128), lambda i: (0, 0)),
        ],
        out_specs=pl.BlockSpec((rows_blk, MODEL_DIM), lambda i: (i, 0)),
        out_shape=jax.ShapeDtypeStruct((N_TOK, MODEL_DIM), jnp.float32),
    )(scale, emb4, q3, proj_W4)


def kernel(token_ids, embed_table, proj_W, bigram_scale):
    b, s = token_ids.shape
    tok_flat = token_ids.astype(jnp.int32).reshape(-1)
    table4 = embed_table.reshape(BIGRAM_VOCAB // PACK, 128)
    emb4, q = _sc_gather(tok_flat, table4)
    q3 = q.reshape(N_TOK // 512, 1, 512)
    proj_W4 = jnp.concatenate([proj_W] * PACK, axis=1)
    scale = bigram_scale.astype(jnp.float32).reshape(1)
    out = _project(emb4, q3, proj_W4, scale)
    return out.reshape(b, s, MODEL_DIM)


# speedup vs baseline: 1.5635x; 1.5599x over previous
import functools

import jax
import jax.numpy as jnp
from jax import lax
from jax.experimental import pallas as pl
from jax.experimental.pallas import tpu as pltpu
from jax.experimental.pallas import tpu_sc as plsc

BIGRAM_VOCAB = 1000000
BIGRAM_DIM = 32
MODEL_DIM = 1024
MOD = BIGRAM_VOCAB - 1
SEQ = 4096
PACK = 128 // BIGRAM_DIM

_info = plsc.get_sparse_core_info()
NC, NS, L = _info.num_cores, _info.num_subcores, _info.num_lanes
NW = NC * NS

N_TOK = 16384
TOK_PER_W = N_TOK // NW
VECS_PER_W = TOK_PER_W // 16
GCHUNK = 128
NGC = TOK_PER_W // GCHUNK


def _sc_hash_gather(tok_hbm, table_hbm, emb_hbm, q_hbm, tok_v, h_v, q_v,
                    rows_v, sem):
    wid = lax.axis_index("s") * NC + lax.axis_index("c")
    base = pl.multiple_of(wid * TOK_PER_W, TOK_PER_W)

    pltpu.sync_copy(tok_hbm.at[pl.ds(base, TOK_PER_W)],
                    tok_v.at[pl.ds(16, TOK_PER_W)])
    prev_off = pl.multiple_of(jnp.maximum(base - 8, 0), 8)
    pltpu.sync_copy(tok_hbm.at[pl.ds(prev_off, 8)], tok_v.at[pl.ds(8, 8)])

    seq_phase = base % SEQ
    lane = lax.iota(jnp.int32, 16)

    for j in range(VECS_PER_W):
        cur = tok_v[pl.ds(16 + 16 * j, 16)]
        prv = tok_v[pl.ds(15 + 16 * j, 16)]
        h = lax.rem(lax.bitwise_xor(cur * 36313, prv * 27191),
                    jnp.int32(MOD))
        if j == 0:
            h = jnp.where(lane + seq_phase == 0, jnp.int32(MOD), h)
        c, r = j // 8, (j % 8) * 16
        lo = lax.bitwise_and(h, 4095)
        h_v[c, pl.ds(r, 16)] = lax.bitwise_or(
            lax.bitwise_and(lax.shift_right_logical(h, 2),
                            jnp.int32(~1023)),
            lax.bitwise_and(h, 1023))
        q_v[pl.ds(16 * j, 16)] = lax.shift_right_logical(lo, 10)

    copies = [
        pltpu.async_copy(table_hbm.at[h_v.at[c]],
                         rows_v.at[pl.ds(c * GCHUNK, GCHUNK), :], sem)
        for c in range(NGC)
    ]
    for cp in copies:
        cp.wait()

    pltpu.sync_copy(rows_v, emb_hbm.at[pl.ds(base, TOK_PER_W), :])
    pltpu.sync_copy(q_v, q_hbm.at[pl.ds(base, TOK_PER_W)])


def _sc_gather(tok_flat, table4):
    mesh = plsc.VectorSubcoreMesh(core_axis_name="c", subcore_axis_name="s")
    k = functools.partial(
        pl.kernel,
        mesh=mesh,
        out_type=(
            jax.ShapeDtypeStruct((N_TOK, 128), jnp.float32),
            jax.ShapeDtypeStruct((N_TOK,), jnp.int32),
        ),
        scratch_types=[
            pltpu.VMEM((TOK_PER_W + 16,), jnp.int32),
            pltpu.VMEM((NGC, GCHUNK), jnp.int32),
            pltpu.VMEM((TOK_PER_W,), jnp.int32),
            pltpu.VMEM((TOK_PER_W, 128), jnp.float32),
            pltpu.SemaphoreType.DMA,
        ],
    )(_sc_hash_gather)
    return k(tok_flat, table4)


REGION = BIGRAM_VOCAB // PACK
RETILE_C = 4096
RETILE_BLOCKS = -(-BIGRAM_VOCAB // RETILE_C)
TABLE4_ROWS = RETILE_BLOCKS * (RETILE_C // PACK)


QROWS = RETILE_C // PACK


def _retile_body(t_ref, out_ref):
    xt = t_ref[...].T
    parts = [
        lax.slice(xt, (QROWS * g, 0), (QROWS * (g + 1), BIGRAM_DIM))
        for g in range(PACK)
    ]
    out_ref[...] = jnp.concatenate(parts, axis=1)


def _retile(table_t):
    return pl.pallas_call(
        _retile_body,
        grid=(RETILE_BLOCKS,),
        in_specs=[
            pl.BlockSpec((BIGRAM_DIM, RETILE_C), lambda i: (0, i)),
        ],
        out_specs=pl.BlockSpec((RETILE_C // PACK, 128), lambda i: (i, 0)),
        out_shape=jax.ShapeDtypeStruct((TABLE4_ROWS, 128), jnp.float32),
    )(table_t)


def _proj_body(scale_ref, emb_ref, q_ref, w_ref, out_ref):
    q = q_ref[0, 0, :]
    grp = lax.broadcasted_iota(jnp.int32, emb_ref.shape, 1) // BIGRAM_DIM
    sel = jnp.where(grp == q[:, None], emb_ref[...], 0.0)
    acc = lax.dot_general(sel, w_ref[...],
                          dimension_numbers=(((1,), (1,)), ((), ())),
                          preferred_element_type=jnp.float32)
    out_ref[...] = acc * scale_ref[0]


def _project(emb4, q3, proj_W4, scale):
    rows_blk = 512
    grid = (N_TOK // rows_blk,)
    return pl.pallas_call(
        _proj_body,
        grid=grid,
        in_specs=[
            pl.BlockSpec(memory_space=pltpu.SMEM),
            pl.BlockSpec((rows_blk, 128), lambda i: (i, 0)),
            pl.BlockSpec((1, 1, rows_blk), lambda i: (i, 0, 0)),
            pl.BlockSpec((MODEL_DIM, 128), lambda i: (0, 0)),
        ],
        out_specs=pl.BlockSpec((rows_blk, MODEL_DIM), lambda i: (i, 0)),
        out_shape=jax.ShapeDtypeStruct((N_TOK, MODEL_DIM), jnp.float32),
    )(scale, emb4, q3, proj_W4)


def kernel(token_ids, embed_table, proj_W, bigram_scale):
    b, s = token_ids.shape
    tok_flat = token_ids.astype(jnp.int32).reshape(-1)
    table4 = _retile(embed_table.T)
    emb4, q = _sc_gather(tok_flat, table4)
    q3 = q.reshape(N_TOK // 512, 1, 512)
    proj_W4 = jnp.concatenate([proj_W] * PACK, axis=1)
    scale = bigram_scale.astype(jnp.float32).reshape(1)
    out = _project(emb4, q3, proj_W4, scale)
    return out.reshape(b, s, MODEL_DIM)


# speedup vs baseline: 1.6910x; 1.0815x over previous
import functools

import jax
import jax.numpy as jnp
from jax import lax
from jax.experimental import pallas as pl
from jax.experimental.pallas import tpu as pltpu
from jax.experimental.pallas import tpu_sc as plsc

BIGRAM_VOCAB = 1000000
BIGRAM_DIM = 32
MODEL_DIM = 1024
MOD = BIGRAM_VOCAB - 1
SEQ = 4096
PACK = 128 // BIGRAM_DIM

_info = plsc.get_sparse_core_info()
NC, NS, L = _info.num_cores, _info.num_subcores, _info.num_lanes
NW = NC * NS

N_TOK = 16384
TOK_PER_W = N_TOK // NW
VECS_PER_W = TOK_PER_W // 16
GCHUNK = 128
NGC = TOK_PER_W // GCHUNK


def _sc_hash_gather(tok_hbm, table_hbm, emb_hbm, q_hbm, tok_v, h_v, q_v,
                    rows_v, sem):
    wid = lax.axis_index("s") * NC + lax.axis_index("c")
    base = pl.multiple_of(wid * TOK_PER_W, TOK_PER_W)

    pltpu.sync_copy(tok_hbm.at[pl.ds(base, TOK_PER_W)],
                    tok_v.at[pl.ds(16, TOK_PER_W)])
    prev_off = pl.multiple_of(jnp.maximum(base - 8, 0), 8)
    pltpu.sync_copy(tok_hbm.at[pl.ds(prev_off, 8)], tok_v.at[pl.ds(8, 8)])

    seq_phase = base % SEQ
    lane = lax.iota(jnp.int32, 16)

    for j in range(VECS_PER_W):
        cur = tok_v[pl.ds(16 + 16 * j, 16)]
        prv = tok_v[pl.ds(15 + 16 * j, 16)]
        h = lax.rem(lax.bitwise_xor(cur * 36313, prv * 27191),
                    jnp.int32(MOD))
        if j == 0:
            h = jnp.where(lane + seq_phase == 0, jnp.int32(MOD), h)
        c, r = j // 8, (j % 8) * 16
        lo = lax.bitwise_and(h, 4095)
        h_v[c, pl.ds(r, 16)] = lax.bitwise_or(
            lax.bitwise_and(lax.shift_right_logical(h, 2),
                            jnp.int32(~1023)),
            lax.bitwise_and(h, 1023))
        q_v[pl.ds(16 * j, 16)] = lax.shift_right_logical(lo, 10)

    copies = [
        pltpu.async_copy(table_hbm.at[h_v.at[c]],
                         rows_v.at[pl.ds(c * GCHUNK, GCHUNK), :], sem)
        for c in range(NGC)
    ]
    for cp in copies:
        cp.wait()

    pltpu.sync_copy(rows_v, emb_hbm.at[pl.ds(base, TOK_PER_W), :])
    pltpu.sync_copy(q_v, q_hbm.at[pl.ds(base, TOK_PER_W)])


def _sc_gather(tok_flat, table4):
    mesh = plsc.VectorSubcoreMesh(core_axis_name="c", subcore_axis_name="s")
    k = functools.partial(
        pl.kernel,
        mesh=mesh,
        out_type=(
            jax.ShapeDtypeStruct((N_TOK, 128), jnp.float32),
            jax.ShapeDtypeStruct((N_TOK,), jnp.int32),
        ),
        scratch_types=[
            pltpu.VMEM((TOK_PER_W + 16,), jnp.int32),
            pltpu.VMEM((NGC, GCHUNK), jnp.int32),
            pltpu.VMEM((TOK_PER_W,), jnp.int32),
            pltpu.VMEM((TOK_PER_W, 128), jnp.float32),
            pltpu.SemaphoreType.DMA,
        ],
    )(_sc_hash_gather)
    return k(tok_flat, table4)


REGION = BIGRAM_VOCAB // PACK
RETILE_C = 4096
RETILE_BLOCKS = -(-BIGRAM_VOCAB // RETILE_C)
TABLE4_ROWS = RETILE_BLOCKS * (RETILE_C // PACK)


QROWS = RETILE_C // PACK


def _retile_body(t_ref, out_ref):
    eye = jnp.eye(BIGRAM_DIM, dtype=jnp.bfloat16)
    xt = lax.dot_general(t_ref[...].astype(jnp.bfloat16), eye,
                         dimension_numbers=(((0,), (0,)), ((), ())),
                         preferred_element_type=jnp.float32)
    parts = [
        lax.slice(xt, (QROWS * g, 0), (QROWS * (g + 1), BIGRAM_DIM))
        for g in range(PACK)
    ]
    out_ref[...] = jnp.concatenate(parts, axis=1)


def _retile(table_t):
    return pl.pallas_call(
        _retile_body,
        grid=(RETILE_BLOCKS,),
        in_specs=[
            pl.BlockSpec((BIGRAM_DIM, RETILE_C), lambda i: (0, i)),
        ],
        out_specs=pl.BlockSpec((RETILE_C // PACK, 128), lambda i: (i, 0)),
        out_shape=jax.ShapeDtypeStruct((TABLE4_ROWS, 128), jnp.float32),
    )(table_t)


def _proj_body(scale_ref, emb_ref, q_ref, w_ref, out_ref):
    q = q_ref[0, 0, :]
    grp = lax.broadcasted_iota(jnp.int32, emb_ref.shape, 1) // BIGRAM_DIM
    sel = jnp.where(grp == q[:, None], emb_ref[...], 0.0)
    acc = lax.dot_general(sel, w_ref[...],
                          dimension_numbers=(((1,), (1,)), ((), ())),
                          preferred_element_type=jnp.float32)
    out_ref[...] = acc * scale_ref[0]


def _project(emb4, q3, proj_W4, scale):
    rows_blk = 512
    grid = (N_TOK // rows_blk,)
    return pl.pallas_call(
        _proj_body,
        grid=grid,
        in_specs=[
            pl.BlockSpec(memory_space=pltpu.SMEM),
            pl.BlockSpec((rows_blk, 128), lambda i: (i, 0)),
            pl.BlockSpec((1, 1, rows_blk), lambda i: (i, 0, 0)),
            pl.BlockSpec((MODEL_DIM, 128), lambda i: (0, 0)),
        ],
        out_specs=pl.BlockSpec((rows_blk, MODEL_DIM), lambda i: (i, 0)),
        out_shape=jax.ShapeDtypeStruct((N_TOK, MODEL_DIM), jnp.float32),
    )(scale, emb4, q3, proj_W4)


def kernel(token_ids, embed_table, proj_W, bigram_scale):
    b, s = token_ids.shape
    tok_flat = token_ids.astype(jnp.int32).reshape(-1)
    table4 = _retile(embed_table.T)
    emb4, q = _sc_gather(tok_flat, table4)
    q3 = q.reshape(N_TOK // 512, 1, 512)
    proj_W4 = jnp.concatenate([proj_W] * PACK, axis=1)
    scale = bigram_scale.astype(jnp.float32).reshape(1)
    out = _project(emb4, q3, proj_W4, scale)
    return out.reshape(b, s, MODEL_DIM)


# speedup vs baseline: 2.2690x; 1.3418x over previous
import functools

import jax
import jax.numpy as jnp
from jax import lax
from jax.experimental import pallas as pl
from jax.experimental.pallas import tpu as pltpu
from jax.experimental.pallas import tpu_sc as plsc

BIGRAM_VOCAB = 1000000
BIGRAM_DIM = 32
MODEL_DIM = 1024
MOD = BIGRAM_VOCAB - 1
SEQ = 4096
PACK = 128 // BIGRAM_DIM

_info = plsc.get_sparse_core_info()
NC, NS, L = _info.num_cores, _info.num_subcores, _info.num_lanes
NW = NC * NS

N_TOK = 16384
TOK_PER_W = N_TOK // NW
VECS_PER_W = TOK_PER_W // 16
GCHUNK = 128
NGC = TOK_PER_W // GCHUNK


def _sc_hash_gather(tok_hbm, table_hbm, emb_hbm, q_hbm, tok_v, h_v, q_v,
                    rows_v, sem):
    wid = lax.axis_index("s") * NC + lax.axis_index("c")
    base = pl.multiple_of(wid * TOK_PER_W, TOK_PER_W)

    pltpu.sync_copy(tok_hbm.at[pl.ds(base, TOK_PER_W)],
                    tok_v.at[pl.ds(16, TOK_PER_W)])
    prev_off = pl.multiple_of(jnp.maximum(base - 8, 0), 8)
    pltpu.sync_copy(tok_hbm.at[pl.ds(prev_off, 8)], tok_v.at[pl.ds(8, 8)])

    seq_phase = base % SEQ
    lane = lax.iota(jnp.int32, 16)

    for j in range(VECS_PER_W):
        cur = tok_v[pl.ds(16 + 16 * j, 16)]
        prv = tok_v[pl.ds(15 + 16 * j, 16)]
        h = lax.rem(lax.bitwise_xor(cur * 36313, prv * 27191),
                    jnp.int32(MOD))
        if j == 0:
            h = jnp.where(lane + seq_phase == 0, jnp.int32(MOD), h)
        c, r = j // 8, (j % 8) * 16
        h_v[c, pl.ds(r, 16)] = lax.bitwise_or(
            lax.bitwise_and(lax.shift_right_logical(h, 2),
                            jnp.int32(~(QROWS - 1))),
            lax.bitwise_and(h, QROWS - 1))
        q_v[pl.ds(16 * j, 16)] = lax.bitwise_and(
            lax.shift_right_logical(h, QROWS.bit_length() - 1), 3)

    copies = [
        pltpu.async_copy(table_hbm.at[h_v.at[c]],
                         rows_v.at[pl.ds(c * GCHUNK, GCHUNK), :], sem)
        for c in range(NGC)
    ]
    for cp in copies:
        cp.wait()

    pltpu.sync_copy(rows_v, emb_hbm.at[pl.ds(base, TOK_PER_W), :])
    pltpu.sync_copy(q_v, q_hbm.at[pl.ds(base, TOK_PER_W)])


def _sc_gather(tok_flat, table4):
    mesh = plsc.VectorSubcoreMesh(core_axis_name="c", subcore_axis_name="s")
    k = functools.partial(
        pl.kernel,
        mesh=mesh,
        out_type=(
            jax.ShapeDtypeStruct((N_TOK, 128), jnp.float32),
            jax.ShapeDtypeStruct((N_TOK,), jnp.int32),
        ),
        scratch_types=[
            pltpu.VMEM((TOK_PER_W + 16,), jnp.int32),
            pltpu.VMEM((NGC, GCHUNK), jnp.int32),
            pltpu.VMEM((TOK_PER_W,), jnp.int32),
            pltpu.VMEM((TOK_PER_W, 128), jnp.float32),
            pltpu.SemaphoreType.DMA,
        ],
    )(_sc_hash_gather)
    return k(tok_flat, table4)


REGION = BIGRAM_VOCAB // PACK
RETILE_C = 16384
RETILE_BLOCKS = -(-BIGRAM_VOCAB // RETILE_C)
TABLE4_ROWS = RETILE_BLOCKS * (RETILE_C // PACK)


QROWS = RETILE_C // PACK


def _retile_body(t_ref, out_ref):
    eye = jnp.eye(BIGRAM_DIM, dtype=jnp.bfloat16)
    xt = lax.dot_general(t_ref[...].astype(jnp.bfloat16), eye,
                         dimension_numbers=(((0,), (0,)), ((), ())),
                         preferred_element_type=jnp.float32)
    parts = [
        lax.slice(xt, (QROWS * g, 0), (QROWS * (g + 1), BIGRAM_DIM))
        for g in range(PACK)
    ]
    out_ref[...] = jnp.concatenate(parts, axis=1)


def _retile(table_t):
    return pl.pallas_call(
        _retile_body,
        grid=(RETILE_BLOCKS,),
        in_specs=[
            pl.BlockSpec((BIGRAM_DIM, RETILE_C), lambda i: (0, i)),
        ],
        out_specs=pl.BlockSpec((RETILE_C // PACK, 128), lambda i: (i, 0)),
        out_shape=jax.ShapeDtypeStruct((TABLE4_ROWS, 128), jnp.float32),
    )(table_t)


def _proj_body(scale_ref, emb_ref, q_ref, w_ref, out_ref):
    q = q_ref[0, 0, :]
    grp = lax.broadcasted_iota(jnp.int32, emb_ref.shape, 1) // BIGRAM_DIM
    sel = jnp.where(grp == q[:, None], emb_ref[...], 0.0)
    acc = lax.dot_general(sel, w_ref[...],
                          dimension_numbers=(((1,), (1,)), ((), ())),
                          preferred_element_type=jnp.float32)
    out_ref[...] = acc * scale_ref[0]


def _project(emb4, q3, proj_W4, scale):
    rows_blk = 512
    grid = (N_TOK // rows_blk,)
    return pl.pallas_call(
        _proj_body,
        grid=grid,
        in_specs=[
            pl.BlockSpec(memory_space=pltpu.SMEM),
            pl.BlockSpec((rows_blk, 128), lambda i: (i, 0)),
            pl.BlockSpec((1, 1, rows_blk), lambda i: (i, 0, 0)),
            pl.BlockSpec((MODEL_DIM, 128), lambda i: (0, 0)),
        ],
        out_specs=pl.BlockSpec((rows_blk, MODEL_DIM), lambda i: (i, 0)),
        out_shape=jax.ShapeDtypeStruct((N_TOK, MODEL_DIM), jnp.float32),
    )(scale, emb4, q3, proj_W4)


def kernel(token_ids, embed_table, proj_W, bigram_scale):
    b, s = token_ids.shape
    tok_flat = token_ids.astype(jnp.int32).reshape(-1)
    table4 = _retile(embed_table.T)
    emb4, q = _sc_gather(tok_flat, table4)
    q3 = q.reshape(N_TOK // 512, 1, 512)
    proj_W4 = jnp.concatenate([proj_W] * PACK, axis=1)
    scale = bigram_scale.astype(jnp.float32).reshape(1)
    out = _project(emb4, q3, proj_W4, scale)
    return out.reshape(b, s, MODEL_DIM)


# speedup vs baseline: 2.2980x; 1.0128x over previous
import functools

import jax
import jax.numpy as jnp
from jax import lax
from jax.experimental import pallas as pl
from jax.experimental.pallas import tpu as pltpu
from jax.experimental.pallas import tpu_sc as plsc

BIGRAM_VOCAB = 1000000
BIGRAM_DIM = 32
MODEL_DIM = 1024
MOD = BIGRAM_VOCAB - 1
SEQ = 4096
PACK = 128 // BIGRAM_DIM

_info = plsc.get_sparse_core_info()
NC, NS, L = _info.num_cores, _info.num_subcores, _info.num_lanes
NW = NC * NS

N_TOK = 16384
TOK_PER_W = N_TOK // NW
VECS_PER_W = TOK_PER_W // 16
GCHUNK = 128
NGC = TOK_PER_W // GCHUNK


def _sc_hash_gather(tok_hbm, table_hbm, emb_hbm, q_hbm, tok_v, h_v, q_v,
                    rows_v, sem):
    wid = lax.axis_index("s") * NC + lax.axis_index("c")
    base = pl.multiple_of(wid * TOK_PER_W, TOK_PER_W)

    pltpu.sync_copy(tok_hbm.at[pl.ds(base, TOK_PER_W)],
                    tok_v.at[pl.ds(16, TOK_PER_W)])
    prev_off = pl.multiple_of(jnp.maximum(base - 8, 0), 8)
    pltpu.sync_copy(tok_hbm.at[pl.ds(prev_off, 8)], tok_v.at[pl.ds(8, 8)])

    seq_phase = base % SEQ
    lane = lax.iota(jnp.int32, 16)

    for j in range(VECS_PER_W):
        cur = tok_v[pl.ds(16 + 16 * j, 16)]
        prv = tok_v[pl.ds(15 + 16 * j, 16)]
        h = lax.rem(lax.bitwise_xor(cur * 36313, prv * 27191),
                    jnp.int32(MOD))
        if j == 0:
            h = jnp.where(lane + seq_phase == 0, jnp.int32(MOD), h)
        c, r = j // 8, (j % 8) * 16
        h_v[c, pl.ds(r, 16)] = lax.bitwise_or(
            lax.bitwise_and(lax.shift_right_logical(h, 2),
                            jnp.int32(~(QROWS - 1))),
            lax.bitwise_and(h, QROWS - 1))
        q_v[pl.ds(16 * j, 16)] = lax.bitwise_and(
            lax.shift_right_logical(h, QROWS.bit_length() - 1), 3)

    copies = [
        pltpu.async_copy(table_hbm.at[h_v.at[c]],
                         rows_v.at[pl.ds(c * GCHUNK, GCHUNK), :], sem)
        for c in range(NGC)
    ]
    for cp in copies:
        cp.wait()

    pltpu.sync_copy(rows_v, emb_hbm.at[pl.ds(base, TOK_PER_W), :])
    pltpu.sync_copy(q_v, q_hbm.at[pl.ds(base, TOK_PER_W)])


def _sc_gather(tok_flat, table4):
    mesh = plsc.VectorSubcoreMesh(core_axis_name="c", subcore_axis_name="s")
    k = functools.partial(
        pl.kernel,
        mesh=mesh,
        out_type=(
            jax.ShapeDtypeStruct((N_TOK, 128), jnp.float32),
            jax.ShapeDtypeStruct((N_TOK,), jnp.int32),
        ),
        scratch_types=[
            pltpu.VMEM((TOK_PER_W + 16,), jnp.int32),
            pltpu.VMEM((NGC, GCHUNK), jnp.int32),
            pltpu.VMEM((TOK_PER_W,), jnp.int32),
            pltpu.VMEM((TOK_PER_W, 128), jnp.float32),
            pltpu.SemaphoreType.DMA,
        ],
    )(_sc_hash_gather)
    return k(tok_flat, table4)


REGION = BIGRAM_VOCAB // PACK
RETILE_C = 32768
RETILE_BLOCKS = -(-BIGRAM_VOCAB // RETILE_C)
TABLE4_ROWS = RETILE_BLOCKS * (RETILE_C // PACK)


QROWS = RETILE_C // PACK


def _retile_body(t_ref, out_ref):
    eye = jnp.eye(BIGRAM_DIM, dtype=jnp.bfloat16)
    xt = lax.dot_general(t_ref[...].astype(jnp.bfloat16), eye,
                         dimension_numbers=(((0,), (0,)), ((), ())),
                         preferred_element_type=jnp.float32)
    parts = [
        lax.slice(xt, (QROWS * g, 0), (QROWS * (g + 1), BIGRAM_DIM))
        for g in range(PACK)
    ]
    out_ref[...] = jnp.concatenate(parts, axis=1)


def _retile(table_t):
    return pl.pallas_call(
        _retile_body,
        grid=(RETILE_BLOCKS,),
        in_specs=[
            pl.BlockSpec((BIGRAM_DIM, RETILE_C), lambda i: (0, i)),
        ],
        out_specs=pl.BlockSpec((RETILE_C // PACK, 128), lambda i: (i, 0)),
        out_shape=jax.ShapeDtypeStruct((TABLE4_ROWS, 128), jnp.float32),
    )(table_t)


def _proj_body(scale_ref, emb_ref, q_ref, w_ref, out_ref):
    q = q_ref[0, 0, :]
    grp = lax.broadcasted_iota(jnp.int32, emb_ref.shape, 1) // BIGRAM_DIM
    sel = jnp.where(grp == q[:, None], emb_ref[...], 0.0)
    acc = lax.dot_general(sel, w_ref[...],
                          dimension_numbers=(((1,), (1,)), ((), ())),
                          preferred_element_type=jnp.float32)
    out_ref[...] = acc * scale_ref[0]


def _project(emb4, q3, proj_W4, scale):
    rows_blk = 512
    grid = (N_TOK // rows_blk,)
    return pl.pallas_call(
        _proj_body,
        grid=grid,
        in_specs=[
            pl.BlockSpec(memory_space=pltpu.SMEM),
            pl.BlockSpec((rows_blk, 128), lambda i: (i, 0)),
            pl.BlockSpec((1, 1, rows_blk), lambda i: (i, 0, 0)),
            pl.BlockSpec((MODEL_DIM, 128), lambda i: (0, 0)),
        ],
        out_specs=pl.BlockSpec((rows_blk, MODEL_DIM), lambda i: (i, 0)),
        out_shape=jax.ShapeDtypeStruct((N_TOK, MODEL_DIM), jnp.float32),
    )(scale, emb4, q3, proj_W4)


def kernel(token_ids, embed_table, proj_W, bigram_scale):
    b, s = token_ids.shape
    tok_flat = token_ids.astype(jnp.int32).reshape(-1)
    table4 = _retile(embed_table.T)
    emb4, q = _sc_gather(tok_flat, table4)
    q3 = q.reshape(N_TOK // 512, 1, 512)
    proj_W4 = jnp.concatenate([proj_W] * PACK, axis=1)
    scale = bigram_scale.astype(jnp.float32).reshape(1)
    out = _project(emb4, q3, proj_W4, scale)
    return out.reshape(b, s, MODEL_DIM)
